# Initial kernel scaffold; baseline (speedup 1.0000x reference)
import jax
import jax.numpy as jnp
from jax.experimental import pallas as pl


def kernel(atom_types, coordinates, adj_list, edge_batch_idx, embed_table, layers, gp0_w, gp0_b, gp1_w, gp1_b, gi0_w, gi0_b, gi1_w, gi1_b):
    raise NotImplementedError("write your pallas kernel here")


# speedup vs baseline: 2.5407x; 2.5407x over previous
import functools

import jax
import jax.numpy as jnp
from jax import lax
from jax.experimental import pallas as pl
from jax.experimental.pallas import tpu as pltpu
from jax.experimental.pallas import tpu_sc as plsc

F32 = jnp.float32
I32 = jnp.int32

NB, NN, NE, DD, VV = 2, 5000, 160000, 128, 32
NNODE = NB * NN
NODE_PAD = 10240
ECHUNK = 128
NCHUNK = 1280
E_PAD = NCHUNK * ECHUNK
NWORK = 32
CH_PER_W = NCHUNK // NWORK
CH_PER_W_SC = NCHUNK // 2 // 16
STRIPE = NODE_PAD // 16

_SC_MESH = plsc.VectorSubcoreMesh(core_axis_name="c", subcore_axis_name="s")


def _sc_gather_coords(ctab, src2d, tgt2d):
    @functools.partial(
        pl.kernel,
        out_type=(
            jax.ShapeDtypeStruct((E_PAD, 16), F32),
            jax.ShapeDtypeStruct((E_PAD, 16), F32),
        ),
        mesh=_SC_MESH,
        scratch_types=[
            pltpu.VMEM((ECHUNK,), I32),
            pltpu.VMEM((ECHUNK,), I32),
            pltpu.VMEM((ECHUNK, 16), F32),
            pltpu.VMEM((ECHUNK, 16), F32),
            pltpu.SemaphoreType.DMA,
            pltpu.SemaphoreType.DMA,
        ],
        compiler_params=pltpu.CompilerParams(use_tc_tiling_on_sc=False),
    )
    def k(ctab_hbm, src_hbm, tgt_hbm, cs_hbm, ct_hbm, is_v, it_v, bs_v, bt_v, sem_s, sem_t):
        wid = lax.axis_index("s") * 2 + lax.axis_index("c")
        c0 = wid * CH_PER_W

        def body(j, carry):
            c = c0 + j
            pltpu.sync_copy(src_hbm.at[c], is_v)
            pltpu.sync_copy(tgt_hbm.at[c], it_v)
            cp_s = pltpu.async_copy(ctab_hbm.at[is_v], bs_v, sem_s)
            cp_t = pltpu.async_copy(ctab_hbm.at[it_v], bt_v, sem_t)
            cp_s.wait()
            cp_t.wait()
            pltpu.sync_copy(bs_v, cs_hbm.at[pl.ds(c * ECHUNK, ECHUNK)])
            pltpu.sync_copy(bt_v, ct_hbm.at[pl.ds(c * ECHUNK, ECHUNK)])
            return carry

        lax.fori_loop(0, CH_PER_W, body, 0)

    return k(ctab, src2d, tgt2d)


def _sc_gather_pair_sum(hps, hpt, src2d, tgt2d):
    @functools.partial(
        pl.kernel,
        out_type=jax.ShapeDtypeStruct((E_PAD, DD), F32),
        mesh=_SC_MESH,
        scratch_types=[
            pltpu.VMEM((ECHUNK,), I32),
            pltpu.VMEM((ECHUNK,), I32),
            pltpu.VMEM((ECHUNK, DD), F32),
            pltpu.VMEM((ECHUNK, DD), F32),
            pltpu.SemaphoreType.DMA,
            pltpu.SemaphoreType.DMA,
        ],
    )
    def k(hps_hbm, hpt_hbm, src_hbm, tgt_hbm, out_hbm, is_v, it_v, bs_v, bt_v, sem_s, sem_t):
        wid = lax.axis_index("s") * 2 + lax.axis_index("c")
        c0 = wid * CH_PER_W

        def body(j, carry):
            c = c0 + j
            pltpu.sync_copy(src_hbm.at[c], is_v)
            pltpu.sync_copy(tgt_hbm.at[c], it_v)
            cp_s = pltpu.async_copy(hps_hbm.at[is_v], bs_v, sem_s)
            cp_t = pltpu.async_copy(hpt_hbm.at[it_v], bt_v, sem_t)
            cp_s.wait()
            cp_t.wait()

            def add_row(r, carry2):
                for q in range(DD // 16):
                    sl = pl.ds(q * 16, 16)
                    bs_v[r, sl] = bs_v[r, sl] + bt_v[r, sl]
                return carry2

            lax.fori_loop(0, ECHUNK, add_row, 0)
            pltpu.sync_copy(bs_v, out_hbm.at[pl.ds(c * ECHUNK, ECHUNK)])
            return carry

        lax.fori_loop(0, CH_PER_W, body, 0)

    return k(hps, hpt, src2d, tgt2d)


def _sc_scatter_add(m, tgt2d):
    @functools.partial(
        pl.kernel,
        out_type=jax.ShapeDtypeStruct((2, NODE_PAD, DD), F32),
        mesh=_SC_MESH,
        scratch_types=[
            pltpu.VMEM_SHARED((NODE_PAD, DD), F32),
            pltpu.VMEM((ECHUNK,), I32),
            pltpu.VMEM((ECHUNK, DD), F32),
        ],
    )
    def k(m_hbm, tgt_hbm, out_hbm, acc_sh, idx_v, m_v):
        cid = lax.axis_index("c")
        sid = lax.axis_index("s")

        def zero_row(r, carry):
            for q in range(DD // 16):
                m_v[r, pl.ds(q * 16, 16)] = jnp.zeros((16,), F32)
            return carry

        lax.fori_loop(0, ECHUNK, zero_row, 0)
        for kk in range(STRIPE // ECHUNK):
            pltpu.sync_copy(m_v, acc_sh.at[pl.ds(sid * STRIPE + kk * ECHUNK, ECHUNK)])
        plsc.subcore_barrier()

        c0 = cid * (NCHUNK // 2) + sid * CH_PER_W_SC

        def body(j, carry):
            c = c0 + j
            pltpu.sync_copy(tgt_hbm.at[c], idx_v)
            pltpu.sync_copy(m_hbm.at[pl.ds(c * ECHUNK, ECHUNK)], m_v)
            pltpu.sync_copy(m_v, acc_sh.at[idx_v], add=True)
            return carry

        lax.fori_loop(0, CH_PER_W_SC, body, 0)
        plsc.subcore_barrier()
        pltpu.sync_copy(
            acc_sh.at[pl.ds(sid * STRIPE, STRIPE)],
            out_hbm.at[cid, pl.ds(sid * STRIPE, STRIPE)],
        )

    return k(m, tgt2d)


def _embed_body(at_ref, tbl_ref, out_ref):
    at = at_ref[...]
    iota = lax.broadcasted_iota(I32, (at.shape[0], VV), 1)
    oh = (at == iota).astype(F32)
    out_ref[...] = jnp.dot(oh, tbl_ref[...], preferred_element_type=F32)


def _tc_embed(at_flat, tbl):
    blk = 2000
    return pl.pallas_call(
        _embed_body,
        grid=(NNODE // blk,),
        in_specs=[
            pl.BlockSpec((blk, 1), lambda i: (i, 0)),
            pl.BlockSpec((VV, DD), lambda i: (0, 0)),
        ],
        out_specs=pl.BlockSpec((blk, DD), lambda i: (i, 0)),
        out_shape=jax.ShapeDtypeStruct((NNODE, DD), F32),
    )(at_flat, tbl)


def _ef_body(cs_ref, ct_ref, ef_ref):
    dvec = ct_ref[...] - cs_ref[...]
    dist2 = jnp.sum(dvec * dvec, axis=1, keepdims=True)
    dist = jnp.sqrt(dist2)
    lane = lax.broadcasted_iota(I32, dvec.shape, 1)
    bz = jnp.sum(jnp.where(lane == 2, dvec, 0.0), axis=1, keepdims=True)
    cosv = jnp.clip(bz / (dist + 1e-8), -1.0 + 1e-6, 1.0 - 1e-6)
    ang = jnp.arctan2(jnp.sqrt(jnp.maximum(1.0 - cosv * cosv, 0.0)), cosv)
    dih = jnp.sqrt(jnp.maximum(dist2 - bz * bz, 0.0))
    bt = 1.0 / (1.0 + jnp.exp(-2.0 * (1.5 - dist)))
    l8 = lax.broadcasted_iota(I32, (dvec.shape[0], 8), 1)
    ef_ref[...] = jnp.where(
        l8 == 0, dist,
        jnp.where(l8 == 1, ang, jnp.where(l8 == 2, dih, jnp.where(l8 == 3, bt, 0.0))))


def _tc_edge_features(cs, ct):
    blk = 4096
    return pl.pallas_call(
        _ef_body,
        grid=(E_PAD // blk,),
        in_specs=[
            pl.BlockSpec((blk, 16), lambda i: (i, 0)),
            pl.BlockSpec((blk, 16), lambda i: (i, 0)),
        ],
        out_specs=pl.BlockSpec((blk, 8), lambda i: (i, 0)),
        out_shape=jax.ShapeDtypeStruct((E_PAD, 8), F32),
    )(cs, ct)


def _proj_body(h_ref, ws_ref, wt_ref, os_ref, ot_ref):
    h = h_ref[...]
    os_ref[...] = jnp.dot(h, ws_ref[...], preferred_element_type=F32)
    ot_ref[...] = jnp.dot(h, wt_ref[...], preferred_element_type=F32)


def _tc_proj(h, ws, wt):
    blk = 2000
    return pl.pallas_call(
        _proj_body,
        grid=(NNODE // blk,),
        in_specs=[
            pl.BlockSpec((blk, DD), lambda i: (i, 0)),
            pl.BlockSpec((DD, DD), lambda i: (0, 0)),
            pl.BlockSpec((DD, DD), lambda i: (0, 0)),
        ],
        out_specs=(
            pl.BlockSpec((blk, DD), lambda i: (i, 0)),
            pl.BlockSpec((blk, DD), lambda i: (i, 0)),
        ),
        out_shape=(
            jax.ShapeDtypeStruct((NNODE, DD), F32),
            jax.ShapeDtypeStruct((NNODE, DD), F32),
        ),
    )(h, ws, wt)


def _edge_mlp_body(ps_ref, ef_ref, we_ref, b0_ref, w1_ref, b1_ref, w2_ref, b2_ref, out_ref):
    x = ps_ref[...] + jnp.dot(ef_ref[...], we_ref[...], preferred_element_type=F32) + b0_ref[...]
    x = jnp.maximum(x, 0.0)
    y = jnp.maximum(jnp.dot(x, w1_ref[...], preferred_element_type=F32) + b1_ref[...], 0.0)
    out_ref[...] = jnp.dot(y, w2_ref[...], preferred_element_type=F32) + b2_ref[...]


def _tc_edge_mlp(psum, ef, we8, b0, w1, b1, w2, b2):
    blk = 2048
    wspec = lambda shape: pl.BlockSpec(shape, lambda i: (0, 0))
    return pl.pallas_call(
        _edge_mlp_body,
        grid=(E_PAD // blk,),
        in_specs=[
            pl.BlockSpec((blk, DD), lambda i: (i, 0)),
            pl.BlockSpec((blk, 8), lambda i: (i, 0)),
            wspec((8, DD)), wspec((1, DD)), wspec((DD, DD)), wspec((1, DD)),
            wspec((DD, DD)), wspec((1, DD)),
        ],
        out_specs=pl.BlockSpec((blk, DD), lambda i: (i, 0)),
        out_shape=jax.ShapeDtypeStruct((E_PAD, DD), F32),
    )(psum, ef, we8, b0, w1, b1, w2, b2)


def _update_body(h_ref, a0_ref, a1_ref, wh_ref, wa_ref, b0_ref, w1_ref, b1_ref,
                 g_ref, bln_ref, out_ref):
    h = h_ref[...]
    a = a0_ref[0] + a1_ref[0]
    u = jnp.dot(h, wh_ref[...], preferred_element_type=F32)
    u = u + jnp.dot(a, wa_ref[...], preferred_element_type=F32) + b0_ref[...]
    u = jnp.maximum(u, 0.0)
    upd = jnp.dot(u, w1_ref[...], preferred_element_type=F32) + b1_ref[...]
    r = upd + h
    mu = jnp.mean(r, axis=-1, keepdims=True)
    c = r - mu
    var = jnp.mean(c * c, axis=-1, keepdims=True)
    out_ref[...] = c * lax.rsqrt(var + 1e-5) * g_ref[...] + bln_ref[...]


def _tc_update(h, aggp, wh, wa, b0, w1, b1, g, bln):
    blk = 1000
    wspec = lambda shape: pl.BlockSpec(shape, lambda i: (0, 0))
    return pl.pallas_call(
        _update_body,
        grid=(NNODE // blk,),
        in_specs=[
            pl.BlockSpec((blk, DD), lambda i: (i, 0)),
            pl.BlockSpec((1, blk, DD), lambda i: (0, i, 0)),
            pl.BlockSpec((1, blk, DD), lambda i: (1, i, 0)),
            wspec((DD, DD)), wspec((DD, DD)), wspec((1, DD)),
            wspec((DD, DD)), wspec((1, DD)), wspec((1, DD)), wspec((1, DD)),
        ],
        out_specs=pl.BlockSpec((blk, DD), lambda i: (i, 0)),
        out_shape=jax.ShapeDtypeStruct((NNODE, DD), F32),
    )(h, aggp, aggp, wh, wa, b0, w1, b1, g, bln)


def _readout_gf_body(h_ref, w0_ref, b0_ref, w1_ref, b1_ref, out_ref):
    h = h_ref[...]
    m0 = jnp.mean(h[:NN], axis=0, keepdims=True)
    m1 = jnp.mean(h[NN:], axis=0, keepdims=True)
    gf = jnp.concatenate([m0, m1], axis=0)
    x = jnp.maximum(jnp.dot(gf, w0_ref[...], preferred_element_type=F32) + b0_ref[...], 0.0)
    out_ref[...] = jnp.dot(x, w1_ref[...], preferred_element_type=F32) + b1_ref[...]


def _tc_readout_gf(h, gp0_w, gp0_b, gp1_w, gp1_b):
    wspec = lambda shape: pl.BlockSpec(shape, lambda: (0, 0))
    return pl.pallas_call(
        _readout_gf_body,
        in_specs=[
            wspec((NNODE, DD)),
            wspec((DD, DD // 2)), wspec((1, DD // 2)),
            wspec((DD // 2, DD // 4)), wspec((1, DD // 4)),
        ],
        out_specs=wspec((NB, DD // 4)),
        out_shape=jax.ShapeDtypeStruct((NB, DD // 4), F32),
    )(h, gp0_w, gp0_b, gp1_w, gp1_b)


def _readout_body(h_ref, gq_ref, wgh_ref, wgg_ref, b0_ref, w1_ref, b1_ref, out_ref):
    b = pl.program_id(0)
    rows = lax.broadcasted_iota(I32, (NB, DD // 4), 0)
    gq = jnp.sum(jnp.where(rows == b, gq_ref[...], 0.0), axis=0, keepdims=True)
    pre = jnp.dot(h_ref[...], wgh_ref[...], preferred_element_type=F32)
    pre = pre + jnp.dot(gq, wgg_ref[...], preferred_element_type=F32) + b0_ref[...]
    pre = jnp.maximum(pre, 0.0)
    out_ref[...] = jnp.dot(pre, w1_ref[...], preferred_element_type=F32) + b1_ref[...]


def _tc_readout(h, gq, wgh, wgg, b0, w1, b1):
    blk = 1000
    wspec = lambda shape: pl.BlockSpec(shape, lambda b, i: (0, 0))
    return pl.pallas_call(
        _readout_body,
        grid=(NB, NN // blk),
        in_specs=[
            pl.BlockSpec((blk, DD), lambda b, i: (b * (NN // blk) + i, 0)),
            pl.BlockSpec((NB, DD // 4), lambda b, i: (0, 0)),
            wspec((DD, DD)), wspec((DD // 4, DD)), wspec((1, DD)),
            wspec((DD, DD)), wspec((1, DD)),
        ],
        out_specs=pl.BlockSpec((blk, DD), lambda b, i: (b * (NN // blk) + i, 0)),
        out_shape=jax.ShapeDtypeStruct((NNODE, DD), F32),
    )(h, gq, wgh, wgg, b0, w1, b1)


def kernel(atom_types, coordinates, adj_list, edge_batch_idx, embed_table, layers,
           gp0_w, gp0_b, gp1_w, gp1_b, gi0_w, gi0_b, gi1_w, gi1_b):
    adj = adj_list.astype(I32)
    eb = edge_batch_idx.astype(I32)
    src = adj[:, 0] + eb * NN
    tgt = adj[:, 1] + eb * NN
    src2d = jnp.concatenate([src, jnp.zeros((E_PAD - NE,), I32)]).reshape(NCHUNK, ECHUNK)
    tgt2d_g = jnp.concatenate([tgt, jnp.zeros((E_PAD - NE,), I32)]).reshape(NCHUNK, ECHUNK)
    tgt2d_s = jnp.concatenate(
        [tgt, jnp.full((E_PAD - NE,), NNODE, I32)]).reshape(NCHUNK, ECHUNK)
    ctab = jnp.pad(coordinates.reshape(NNODE, 3).astype(F32), ((0, 0), (0, 13)))
    at_flat = atom_types.astype(I32).reshape(NNODE, 1)

    cs, ct = _sc_gather_coords(ctab, src2d, tgt2d_g)
    ef = _tc_edge_features(cs, ct)

    h = _tc_embed(at_flat, embed_table.astype(F32))

    row = lambda v: v.reshape(1, -1).astype(F32)
    for p in layers:
        ws = p['m0_w'][:DD]
        wt = p['m0_w'][DD:2 * DD]
        we8 = jnp.pad(p['m0_w'][2 * DD:2 * DD + 4], ((0, 4), (0, 0)))
        hps, hpt = _tc_proj(h, ws, wt)
        psum = _sc_gather_pair_sum(hps, hpt, src2d, tgt2d_g)
        m = _tc_edge_mlp(psum, ef, we8, row(p['m0_b']), p['m1_w'], row(p['m1_b']),
                         p['m2_w'], row(p['m2_b']))
        aggp = _sc_scatter_add(m, tgt2d_s)
        h = _tc_update(h, aggp, p['u0_w'][:DD], p['u0_w'][DD:], row(p['u0_b']),
                       p['u1_w'], row(p['u1_b']), row(p['ln_g']), row(p['ln_b']))

    gq = _tc_readout_gf(h, gp0_w, row(gp0_b), gp1_w, row(gp1_b))
    out = _tc_readout(h, gq, gi0_w[:DD], gi0_w[DD:], row(gi0_b), gi1_w, row(gi1_b))
    return out.reshape(NB, NN, DD)


# speedup vs baseline: 2.9146x; 1.1472x over previous
import functools

import jax
import jax.numpy as jnp
from jax import lax
from jax.experimental import pallas as pl
from jax.experimental.pallas import tpu as pltpu
from jax.experimental.pallas import tpu_sc as plsc

F32 = jnp.float32
I32 = jnp.int32

NB, NN, NE, DD, VV = 2, 5000, 160000, 128, 32
NNODE = NB * NN
NODE_PAD = 10240
ECHUNK = 128
NCHUNK = 1280
E_PAD = NCHUNK * ECHUNK
NWORK = 32
CH_PER_W = NCHUNK // NWORK
CH_PER_W_SC = NCHUNK // 2 // 16
STRIPE = NODE_PAD // 16

_SC_MESH = plsc.VectorSubcoreMesh(core_axis_name="c", subcore_axis_name="s")


def _sc_gather_coords(ctab, src2d, tgt2d):
    @functools.partial(
        pl.kernel,
        out_type=(
            jax.ShapeDtypeStruct((E_PAD, 16), F32),
            jax.ShapeDtypeStruct((E_PAD, 16), F32),
        ),
        mesh=_SC_MESH,
        scratch_types=[
            pltpu.VMEM((ECHUNK,), I32),
            pltpu.VMEM((ECHUNK,), I32),
            pltpu.VMEM((ECHUNK, 16), F32),
            pltpu.VMEM((ECHUNK, 16), F32),
            pltpu.SemaphoreType.DMA,
            pltpu.SemaphoreType.DMA,
        ],
        compiler_params=pltpu.CompilerParams(use_tc_tiling_on_sc=False),
    )
    def k(ctab_hbm, src_hbm, tgt_hbm, cs_hbm, ct_hbm, is_v, it_v, bs_v, bt_v, sem_s, sem_t):
        wid = lax.axis_index("s") * 2 + lax.axis_index("c")
        c0 = wid * CH_PER_W

        def body(j, carry):
            c = c0 + j
            pltpu.sync_copy(src_hbm.at[c], is_v)
            pltpu.sync_copy(tgt_hbm.at[c], it_v)
            cp_s = pltpu.async_copy(ctab_hbm.at[is_v], bs_v, sem_s)
            cp_t = pltpu.async_copy(ctab_hbm.at[it_v], bt_v, sem_t)
            cp_s.wait()
            cp_t.wait()
            pltpu.sync_copy(bs_v, cs_hbm.at[pl.ds(c * ECHUNK, ECHUNK)])
            pltpu.sync_copy(bt_v, ct_hbm.at[pl.ds(c * ECHUNK, ECHUNK)])
            return carry

        lax.fori_loop(0, CH_PER_W, body, 0)

    return k(ctab, src2d, tgt2d)


def _sc_gather_pair_sum(hps, hpt, src2d, tgt2d):
    @functools.partial(
        pl.kernel,
        out_type=jax.ShapeDtypeStruct((E_PAD, DD), F32),
        mesh=_SC_MESH,
        scratch_types=[
            pltpu.VMEM((ECHUNK,), I32),
            pltpu.VMEM((ECHUNK,), I32),
            pltpu.VMEM((ECHUNK,), I32),
            pltpu.VMEM((ECHUNK,), I32),
            pltpu.VMEM((ECHUNK, DD), F32),
            pltpu.VMEM((ECHUNK, DD), F32),
            pltpu.VMEM((ECHUNK, DD), F32),
            pltpu.VMEM((ECHUNK, DD), F32),
            pltpu.SemaphoreType.DMA,
            pltpu.SemaphoreType.DMA,
        ],
    )
    def k(hps_hbm, hpt_hbm, src_hbm, tgt_hbm, out_hbm,
          is0, it0, is1, it1, bs0, bt0, bs1, bt1, sem0, sem1):
        wid = lax.axis_index("s") * 2 + lax.axis_index("c")
        c0 = wid * CH_PER_W
        isv, itv = (is0, is1), (it0, it1)
        bsv, btv = (bs0, bs1), (bt0, bt1)
        sems = (sem0, sem1)

        def prefetch(b, c):
            pltpu.sync_copy(src_hbm.at[c], isv[b])
            pltpu.sync_copy(tgt_hbm.at[c], itv[b])
            pltpu.async_copy(hps_hbm.at[isv[b]], bsv[b], sems[b])
            pltpu.async_copy(hpt_hbm.at[itv[b]], btv[b], sems[b])

        prefetch(0, c0)

        @pl.loop(0, CH_PER_W, step=2)
        def outer(g):
            for b in range(2):
                j = g + b
                c = c0 + j

                @pl.when(j + 1 < CH_PER_W)
                def _():
                    prefetch(b ^ 1, c + 1)

                pltpu.make_async_copy(hps_hbm.at[isv[b]], bsv[b], sems[b]).wait()
                pltpu.make_async_copy(hpt_hbm.at[itv[b]], btv[b], sems[b]).wait()

                def add_row(r, carry2):
                    for q in range(DD // 16):
                        sl = pl.ds(q * 16, 16)
                        plsc.addupdate(bsv[b].at[r, sl], btv[b][r, sl])
                    return carry2

                lax.fori_loop(0, ECHUNK, add_row, 0)
                pltpu.sync_copy(bsv[b], out_hbm.at[pl.ds(c * ECHUNK, ECHUNK)])

    return k(hps, hpt, src2d, tgt2d)


def _sc_scatter_add(m, tgt2d):
    @functools.partial(
        pl.kernel,
        out_type=jax.ShapeDtypeStruct((2, NODE_PAD, DD), F32),
        mesh=_SC_MESH,
        scratch_types=[
            pltpu.VMEM_SHARED((NODE_PAD, DD), F32),
            pltpu.VMEM((ECHUNK,), I32),
            pltpu.VMEM((ECHUNK,), I32),
            pltpu.VMEM((ECHUNK, DD), F32),
            pltpu.VMEM((ECHUNK, DD), F32),
            pltpu.SemaphoreType.DMA,
            pltpu.SemaphoreType.DMA,
        ],
    )
    def k(m_hbm, tgt_hbm, out_hbm, acc_sh, id0, id1, m0, m1, sem0, sem1):
        cid = lax.axis_index("c")
        sid = lax.axis_index("s")
        idv, mv, sems = (id0, id1), (m0, m1), (sem0, sem1)

        def zero_row(r, carry):
            for q in range(DD // 16):
                m0[r, pl.ds(q * 16, 16)] = jnp.zeros((16,), F32)
            return carry

        lax.fori_loop(0, ECHUNK, zero_row, 0)
        for kk in range(STRIPE // ECHUNK):
            pltpu.sync_copy(m0, acc_sh.at[pl.ds(sid * STRIPE + kk * ECHUNK, ECHUNK)])
        plsc.subcore_barrier()

        c0 = cid * (NCHUNK // 2) + sid * CH_PER_W_SC

        def prefetch(b, c):
            pltpu.sync_copy(tgt_hbm.at[c], idv[b])
            pltpu.async_copy(m_hbm.at[pl.ds(c * ECHUNK, ECHUNK)], mv[b], sems[b])

        prefetch(0, c0)

        @pl.loop(0, CH_PER_W_SC, step=2)
        def outer(g):
            for b in range(2):
                j = g + b
                c = c0 + j
                pltpu.make_async_copy(
                    m_hbm.at[pl.ds(c * ECHUNK, ECHUNK)], mv[b], sems[b]).wait()

                @pl.when(j + 1 < CH_PER_W_SC)
                def _():
                    prefetch(b ^ 1, c + 1)

                pltpu.sync_copy(mv[b], acc_sh.at[idv[b]], add=True)

        plsc.subcore_barrier()
        pltpu.sync_copy(
            acc_sh.at[pl.ds(sid * STRIPE, STRIPE)],
            out_hbm.at[cid, pl.ds(sid * STRIPE, STRIPE)],
        )

    return k(m, tgt2d)


def _embed_body(at_ref, tbl_ref, out_ref):
    at = at_ref[...]
    iota = lax.broadcasted_iota(I32, (at.shape[0], VV), 1)
    oh = (at == iota).astype(F32)
    out_ref[...] = jnp.dot(oh, tbl_ref[...], preferred_element_type=F32)


def _tc_embed(at_flat, tbl):
    blk = 2000
    return pl.pallas_call(
        _embed_body,
        grid=(NNODE // blk,),
        in_specs=[
            pl.BlockSpec((blk, 1), lambda i: (i, 0)),
            pl.BlockSpec((VV, DD), lambda i: (0, 0)),
        ],
        out_specs=pl.BlockSpec((blk, DD), lambda i: (i, 0)),
        out_shape=jax.ShapeDtypeStruct((NNODE, DD), F32),
    )(at_flat, tbl)


def _ef_body(cs_ref, ct_ref, ef_ref):
    dvec = ct_ref[...] - cs_ref[...]
    dist2 = jnp.sum(dvec * dvec, axis=1, keepdims=True)
    dist = jnp.sqrt(dist2)
    lane = lax.broadcasted_iota(I32, dvec.shape, 1)
    bz = jnp.sum(jnp.where(lane == 2, dvec, 0.0), axis=1, keepdims=True)
    cosv = jnp.clip(bz / (dist + 1e-8), -1.0 + 1e-6, 1.0 - 1e-6)
    ang = jnp.arctan2(jnp.sqrt(jnp.maximum(1.0 - cosv * cosv, 0.0)), cosv)
    dih = jnp.sqrt(jnp.maximum(dist2 - bz * bz, 0.0))
    bt = 1.0 / (1.0 + jnp.exp(-2.0 * (1.5 - dist)))
    l8 = lax.broadcasted_iota(I32, (dvec.shape[0], 8), 1)
    ef_ref[...] = jnp.where(
        l8 == 0, dist,
        jnp.where(l8 == 1, ang, jnp.where(l8 == 2, dih, jnp.where(l8 == 3, bt, 0.0))))


def _tc_edge_features(cs, ct):
    blk = 4096
    return pl.pallas_call(
        _ef_body,
        grid=(E_PAD // blk,),
        in_specs=[
            pl.BlockSpec((blk, 16), lambda i: (i, 0)),
            pl.BlockSpec((blk, 16), lambda i: (i, 0)),
        ],
        out_specs=pl.BlockSpec((blk, 8), lambda i: (i, 0)),
        out_shape=jax.ShapeDtypeStruct((E_PAD, 8), F32),
    )(cs, ct)


def _proj_body(h_ref, ws_ref, wt_ref, os_ref, ot_ref):
    h = h_ref[...]
    os_ref[...] = jnp.dot(h, ws_ref[...], preferred_element_type=F32)
    ot_ref[...] = jnp.dot(h, wt_ref[...], preferred_element_type=F32)


def _tc_proj(h, ws, wt):
    blk = 2000
    return pl.pallas_call(
        _proj_body,
        grid=(NNODE // blk,),
        in_specs=[
            pl.BlockSpec((blk, DD), lambda i: (i, 0)),
            pl.BlockSpec((DD, DD), lambda i: (0, 0)),
            pl.BlockSpec((DD, DD), lambda i: (0, 0)),
        ],
        out_specs=(
            pl.BlockSpec((blk, DD), lambda i: (i, 0)),
            pl.BlockSpec((blk, DD), lambda i: (i, 0)),
        ),
        out_shape=(
            jax.ShapeDtypeStruct((NNODE, DD), F32),
            jax.ShapeDtypeStruct((NNODE, DD), F32),
        ),
    )(h, ws, wt)


def _edge_mlp_body(ps_ref, ef_ref, we_ref, b0_ref, w1_ref, b1_ref, w2_ref, b2_ref, out_ref):
    x = ps_ref[...] + jnp.dot(ef_ref[...], we_ref[...], preferred_element_type=F32) + b0_ref[...]
    x = jnp.maximum(x, 0.0)
    y = jnp.maximum(jnp.dot(x, w1_ref[...], preferred_element_type=F32) + b1_ref[...], 0.0)
    out_ref[...] = jnp.dot(y, w2_ref[...], preferred_element_type=F32) + b2_ref[...]


def _tc_edge_mlp(psum, ef, we8, b0, w1, b1, w2, b2):
    blk = 2048
    wspec = lambda shape: pl.BlockSpec(shape, lambda i: (0, 0))
    return pl.pallas_call(
        _edge_mlp_body,
        grid=(E_PAD // blk,),
        in_specs=[
            pl.BlockSpec((blk, DD), lambda i: (i, 0)),
            pl.BlockSpec((blk, 8), lambda i: (i, 0)),
            wspec((8, DD)), wspec((1, DD)), wspec((DD, DD)), wspec((1, DD)),
            wspec((DD, DD)), wspec((1, DD)),
        ],
        out_specs=pl.BlockSpec((blk, DD), lambda i: (i, 0)),
        out_shape=jax.ShapeDtypeStruct((E_PAD, DD), F32),
    )(psum, ef, we8, b0, w1, b1, w2, b2)


def _update_body(h_ref, a0_ref, a1_ref, wh_ref, wa_ref, b0_ref, w1_ref, b1_ref,
                 g_ref, bln_ref, out_ref):
    h = h_ref[...]
    a = a0_ref[0] + a1_ref[0]
    u = jnp.dot(h, wh_ref[...], preferred_element_type=F32)
    u = u + jnp.dot(a, wa_ref[...], preferred_element_type=F32) + b0_ref[...]
    u = jnp.maximum(u, 0.0)
    upd = jnp.dot(u, w1_ref[...], preferred_element_type=F32) + b1_ref[...]
    r = upd + h
    mu = jnp.mean(r, axis=-1, keepdims=True)
    c = r - mu
    var = jnp.mean(c * c, axis=-1, keepdims=True)
    out_ref[...] = c * lax.rsqrt(var + 1e-5) * g_ref[...] + bln_ref[...]


def _tc_update(h, aggp, wh, wa, b0, w1, b1, g, bln):
    blk = 1000
    wspec = lambda shape: pl.BlockSpec(shape, lambda i: (0, 0))
    return pl.pallas_call(
        _update_body,
        grid=(NNODE // blk,),
        in_specs=[
            pl.BlockSpec((blk, DD), lambda i: (i, 0)),
            pl.BlockSpec((1, blk, DD), lambda i: (0, i, 0)),
            pl.BlockSpec((1, blk, DD), lambda i: (1, i, 0)),
            wspec((DD, DD)), wspec((DD, DD)), wspec((1, DD)),
            wspec((DD, DD)), wspec((1, DD)), wspec((1, DD)), wspec((1, DD)),
        ],
        out_specs=pl.BlockSpec((blk, DD), lambda i: (i, 0)),
        out_shape=jax.ShapeDtypeStruct((NNODE, DD), F32),
    )(h, aggp, aggp, wh, wa, b0, w1, b1, g, bln)


def _readout_gf_body(h_ref, w0_ref, b0_ref, w1_ref, b1_ref, out_ref):
    h = h_ref[...]
    m0 = jnp.mean(h[:NN], axis=0, keepdims=True)
    m1 = jnp.mean(h[NN:], axis=0, keepdims=True)
    gf = jnp.concatenate([m0, m1], axis=0)
    x = jnp.maximum(jnp.dot(gf, w0_ref[...], preferred_element_type=F32) + b0_ref[...], 0.0)
    out_ref[...] = jnp.dot(x, w1_ref[...], preferred_element_type=F32) + b1_ref[...]


def _tc_readout_gf(h, gp0_w, gp0_b, gp1_w, gp1_b):
    wspec = lambda shape: pl.BlockSpec(shape, lambda: (0, 0))
    return pl.pallas_call(
        _readout_gf_body,
        in_specs=[
            wspec((NNODE, DD)),
            wspec((DD, DD // 2)), wspec((1, DD // 2)),
            wspec((DD // 2, DD // 4)), wspec((1, DD // 4)),
        ],
        out_specs=wspec((NB, DD // 4)),
        out_shape=jax.ShapeDtypeStruct((NB, DD // 4), F32),
    )(h, gp0_w, gp0_b, gp1_w, gp1_b)


def _readout_body(h_ref, gq_ref, wgh_ref, wgg_ref, b0_ref, w1_ref, b1_ref, out_ref):
    b = pl.program_id(0)
    rows = lax.broadcasted_iota(I32, (NB, DD // 4), 0)
    gq = jnp.sum(jnp.where(rows == b, gq_ref[...], 0.0), axis=0, keepdims=True)
    pre = jnp.dot(h_ref[...], wgh_ref[...], preferred_element_type=F32)
    pre = pre + jnp.dot(gq, wgg_ref[...], preferred_element_type=F32) + b0_ref[...]
    pre = jnp.maximum(pre, 0.0)
    out_ref[...] = jnp.dot(pre, w1_ref[...], preferred_element_type=F32) + b1_ref[...]


def _tc_readout(h, gq, wgh, wgg, b0, w1, b1):
    blk = 1000
    wspec = lambda shape: pl.BlockSpec(shape, lambda b, i: (0, 0))
    return pl.pallas_call(
        _readout_body,
        grid=(NB, NN // blk),
        in_specs=[
            pl.BlockSpec((blk, DD), lambda b, i: (b * (NN // blk) + i, 0)),
            pl.BlockSpec((NB, DD // 4), lambda b, i: (0, 0)),
            wspec((DD, DD)), wspec((DD // 4, DD)), wspec((1, DD)),
            wspec((DD, DD)), wspec((1, DD)),
        ],
        out_specs=pl.BlockSpec((blk, DD), lambda b, i: (b * (NN // blk) + i, 0)),
        out_shape=jax.ShapeDtypeStruct((NNODE, DD), F32),
    )(h, gq, wgh, wgg, b0, w1, b1)


def kernel(atom_types, coordinates, adj_list, edge_batch_idx, embed_table, layers,
           gp0_w, gp0_b, gp1_w, gp1_b, gi0_w, gi0_b, gi1_w, gi1_b):
    adj = adj_list.astype(I32)
    eb = edge_batch_idx.astype(I32)
    src = adj[:, 0] + eb * NN
    tgt = adj[:, 1] + eb * NN
    src2d = jnp.concatenate([src, jnp.zeros((E_PAD - NE,), I32)]).reshape(NCHUNK, ECHUNK)
    tgt2d_g = jnp.concatenate([tgt, jnp.zeros((E_PAD - NE,), I32)]).reshape(NCHUNK, ECHUNK)
    tgt2d_s = jnp.concatenate(
        [tgt, jnp.full((E_PAD - NE,), NNODE, I32)]).reshape(NCHUNK, ECHUNK)
    ctab = jnp.pad(coordinates.reshape(NNODE, 3).astype(F32), ((0, 0), (0, 13)))
    at_flat = atom_types.astype(I32).reshape(NNODE, 1)

    cs, ct = _sc_gather_coords(ctab, src2d, tgt2d_g)
    ef = _tc_edge_features(cs, ct)

    h = _tc_embed(at_flat, embed_table.astype(F32))

    row = lambda v: v.reshape(1, -1).astype(F32)
    for p in layers:
        ws = p['m0_w'][:DD]
        wt = p['m0_w'][DD:2 * DD]
        we8 = jnp.pad(p['m0_w'][2 * DD:2 * DD + 4], ((0, 4), (0, 0)))
        hps, hpt = _tc_proj(h, ws, wt)
        psum = _sc_gather_pair_sum(hps, hpt, src2d, tgt2d_g)
        m = _tc_edge_mlp(psum, ef, we8, row(p['m0_b']), p['m1_w'], row(p['m1_b']),
                         p['m2_w'], row(p['m2_b']))
        aggp = _sc_scatter_add(m, tgt2d_s)
        h = _tc_update(h, aggp, p['u0_w'][:DD], p['u0_w'][DD:], row(p['u0_b']),
                       p['u1_w'], row(p['u1_b']), row(p['ln_g']), row(p['ln_b']))

    gq = _tc_readout_gf(h, gp0_w, row(gp0_b), gp1_w, row(gp1_b))
    out = _tc_readout(h, gq, gi0_w[:DD], gi0_w[DD:], row(gi0_b), gi1_w, row(gi1_b))
    return out.reshape(NB, NN, DD)
